# baseline (speedup 1.0000x reference)
import functools

import jax
import jax.numpy as jnp
import numpy as np
from jax import lax
from jax.experimental import pallas as pl
from jax.experimental.pallas import tpu as pltpu
from jax.experimental.pallas import tpu_sc as plsc

N = 10000
K = 16
NUM_ATOMS = 4
NPTS = 2 * NUM_ATOMS
NPAIR = NPTS * NPTS
FDIM = 2 * NPAIR
DIST_EPS = 0.01

NUM_CORES = 2
NUM_SUBCORES = 16
NW = NUM_CORES * NUM_SUBCORES
CHUNK = 128
CHUNKS_PER_W = 40
LAST_CHUNKS = N * K // CHUNK - (NW - 1) * CHUNKS_PER_W
ROWS_PER_W = CHUNKS_PER_W * CHUNK

NB = 200
EB = NB * K
GRID = N // NB


def _pair_matrices():
    wt = np.zeros((16, 3 * NPAIR), np.float32)
    wb = np.zeros((16, 3 * NPAIR), np.float32)
    for p in range(NPTS):
        for q in range(NPTS):
            for c in range(3):
                col = c * NPAIR + p * NPTS + q
                if p < 4:
                    wt[p * 3 + c, col] += 1.0
                else:
                    wb[(p - 4) * 3 + c, col] += 1.0
                if q < 4:
                    wt[q * 3 + c, col] -= 1.0
                else:
                    wb[(q - 4) * 3 + c, col] -= 1.0
    s = np.zeros((3 * NPAIR, FDIM), np.float32)
    for c in range(3):
        for l in range(NPAIR):
            s[c * NPAIR + l, l] = 1.0
            s[c * NPAIR + l, NPAIR + l] = 1.0
    wext = np.zeros((144, 256), np.float32)
    for j in range(8):
        wext[16 * j:16 * j + 16, :192] = wb
        wext[16 * j + 12, 192] = 1.0
    wext[128:144, :192] = wt
    wext[128 + 12, 193] = 1.0
    return wext, s


_WEXT, _S = _pair_matrices()
_SEL = np.zeros((8, 128), np.float32)
for _j in range(8):
    _SEL[_j, 16 * _j:16 * _j + 16] = 1.0


def _sc_gather(table: jax.Array, idx2d: jax.Array) -> jax.Array:
    mesh = plsc.VectorSubcoreMesh(core_axis_name="c", subcore_axis_name="s")

    @functools.partial(
        pl.kernel,
        out_type=jax.ShapeDtypeStruct((N * K, 16), jnp.float32),
        mesh=mesh,
        scratch_types=[
            pltpu.VMEM((CHUNKS_PER_W, CHUNK), jnp.int32),
            pltpu.VMEM((ROWS_PER_W, 16), jnp.float32),
            pltpu.SemaphoreType.DMA,
        ],
        compiler_params=pltpu.CompilerParams(use_tc_tiling_on_sc=False),
    )
    def gather_kernel(table_hbm, idx_hbm, out_hbm, idx_v, rows_v, sem):
        wid = lax.axis_index("s") * NUM_CORES + lax.axis_index("c")

        def run(n_chunks):
            pltpu.sync_copy(
                idx_hbm.at[pl.ds(wid * CHUNKS_PER_W, n_chunks)],
                idx_v.at[pl.ds(0, n_chunks)])

            @pl.loop(0, n_chunks)
            def _fire(j):
                pltpu.async_copy(
                    table_hbm.at[idx_v.at[j]],
                    rows_v.at[pl.ds(j * CHUNK, CHUNK)],
                    sem,
                )

            @pl.loop(0, n_chunks)
            def _drain(j):
                pltpu.make_async_copy(
                    table_hbm.at[idx_v.at[j]],
                    rows_v.at[pl.ds(j * CHUNK, CHUNK)],
                    sem,
                ).wait()

            pltpu.sync_copy(
                rows_v.at[pl.ds(0, n_chunks * CHUNK)],
                out_hbm.at[pl.ds(wid * ROWS_PER_W, n_chunks * CHUNK)])

        @pl.when(wid < NW - 1)
        def _full():
            run(CHUNKS_PER_W)

        @pl.when(wid == NW - 1)
        def _last():
            run(LAST_CHUNKS)

    return gather_kernel(table, idx2d)


def _featurize_body(t_ref, g_ref, sel_ref, we_ref, s_ref, o_ref):
    t = t_ref[...]
    g2 = g_ref[...]
    g_b = jnp.broadcast_to(g2[:, None, :], (EB // 8, 8, 128)).reshape(EB, 128)
    sel = jnp.broadcast_to(sel_ref[...][None, :, :], (EB // 8, 8, 128)).reshape(
        EB, 128)
    t_b = jnp.broadcast_to(t[:, None, :], (NB, K, 16)).reshape(EB, 16)
    u = jnp.concatenate([g_b * sel, t_b], axis=1)
    te = jax.lax.dot(u, we_ref[...])
    tt = te[:, :192]
    d2w = jax.lax.dot(tt * tt, s_ref[...])
    d = jnp.sqrt(d2w + 1e-6) + DIST_EPS
    m = te[:, 192:193] * te[:, 193:194]
    lanes = jax.lax.broadcasted_iota(jnp.int32, (EB, FDIM), 1)
    feat = jnp.where(lanes < NPAIR, jnp.log(d), 1.0 / d)
    o_ref[...] = feat * m


def _tc_featurize(table: jax.Array, gath: jax.Array) -> jax.Array:
    return pl.pallas_call(
        _featurize_body,
        grid=(GRID,),
        in_specs=[
            pl.BlockSpec((NB, 16), lambda i: (i, 0)),
            pl.BlockSpec((EB // 8, 128), lambda i: (i, 0)),
            pl.BlockSpec((8, 128), lambda i: (0, 0)),
            pl.BlockSpec((144, 256), lambda i: (0, 0)),
            pl.BlockSpec((3 * NPAIR, FDIM), lambda i: (0, 0)),
        ],
        out_specs=pl.BlockSpec((EB, FDIM), lambda i: (i, 0)),
        out_shape=jax.ShapeDtypeStruct((N * K, FDIM), jnp.float32),
        compiler_params=pltpu.CompilerParams(
            dimension_semantics=("arbitrary",),
        ),
    )(table, gath, jnp.asarray(_SEL), jnp.asarray(_WEXT), jnp.asarray(_S))


def kernel(X, edge_idx, C):
    b, n, k = edge_idx.shape
    xf = X.reshape(n, NUM_ATOMS * 3)
    mask = (C.reshape(n) > 0).astype(jnp.float32)
    table = jnp.concatenate(
        [xf, mask[:, None], jnp.zeros((n, 3), jnp.float32)], axis=1)
    table_pad = jnp.pad(table, ((0, 0), (0, 112)))
    idx2d = (edge_idx.reshape(n * k // CHUNK, CHUNK) * 8).astype(jnp.int32)
    gath = _sc_gather(table_pad.reshape(8 * n, 16), idx2d)
    feat = _tc_featurize(table, gath.reshape(n * k // 8, 128))
    return feat.reshape(b, n, k, FDIM)


# speedup vs baseline: 1.0070x; 1.0070x over previous
import functools

import jax
import jax.numpy as jnp
import numpy as np
from jax import lax
from jax.experimental import pallas as pl
from jax.experimental.pallas import tpu as pltpu
from jax.experimental.pallas import tpu_sc as plsc

N = 10000
K = 16
NUM_ATOMS = 4
NPTS = 2 * NUM_ATOMS
NPAIR = NPTS * NPTS
FDIM = 2 * NPAIR
DIST_EPS = 0.01

NUM_CORES = 2
NUM_SUBCORES = 16
NW = NUM_CORES * NUM_SUBCORES
CHUNK = 128
CHUNKS_PER_W = 40
LAST_CHUNKS = N * K // CHUNK - (NW - 1) * CHUNKS_PER_W
ROWS_PER_W = CHUNKS_PER_W * CHUNK

NB = 200
EB = NB * K
GRID = N // NB


def _pair_matrices():
    wt = np.zeros((16, 3 * NPAIR), np.float32)
    wb = np.zeros((16, 3 * NPAIR), np.float32)
    for p in range(NPTS):
        for q in range(NPTS):
            for c in range(3):
                col = c * NPAIR + p * NPTS + q
                if p < 4:
                    wt[p * 3 + c, col] += 1.0
                else:
                    wb[(p - 4) * 3 + c, col] += 1.0
                if q < 4:
                    wt[q * 3 + c, col] -= 1.0
                else:
                    wb[(q - 4) * 3 + c, col] -= 1.0
    s = np.zeros((3 * NPAIR, FDIM), np.float32)
    for c in range(3):
        for l in range(NPAIR):
            s[c * NPAIR + l, l] = 1.0
            s[c * NPAIR + l, NPAIR + l] = 1.0
    wext = np.zeros((144, 256), np.float32)
    for j in range(8):
        wext[16 * j:16 * j + 16, :192] = wb
        wext[16 * j + 12, 192] = 1.0
    wext[128:144, :192] = wt
    wext[128 + 12, 193] = 1.0
    return wext, s


_WEXT, _S = _pair_matrices()
_SEL = np.zeros((8, 128), np.float32)
for _j in range(8):
    _SEL[_j, 16 * _j:16 * _j + 16] = 1.0


def _sc_gather(table: jax.Array, idx2d: jax.Array) -> jax.Array:
    mesh = plsc.VectorSubcoreMesh(core_axis_name="c", subcore_axis_name="s")

    @functools.partial(
        pl.kernel,
        out_type=jax.ShapeDtypeStruct((N * K, 16), jnp.float32),
        mesh=mesh,
        scratch_types=[
            pltpu.VMEM((CHUNKS_PER_W, CHUNK), jnp.int32),
            pltpu.VMEM((ROWS_PER_W, 16), jnp.float32),
            pltpu.SemaphoreType.DMA,
        ],
        compiler_params=pltpu.CompilerParams(use_tc_tiling_on_sc=False),
    )
    def gather_kernel(table_hbm, idx_hbm, out_hbm, idx_v, rows_v, sem):
        wid = lax.axis_index("s") * NUM_CORES + lax.axis_index("c")

        def run(n_chunks):
            pltpu.sync_copy(
                idx_hbm.at[pl.ds(wid * CHUNKS_PER_W, n_chunks)],
                idx_v.at[pl.ds(0, n_chunks)])

            @pl.loop(0, n_chunks)
            def _scale(j):
                for l in range(8):
                    idx_v[j, pl.ds(l * 16, 16)] = idx_v[j, pl.ds(l * 16, 16)] * 8

            @pl.loop(0, n_chunks)
            def _fire(j):
                pltpu.async_copy(
                    table_hbm.at[idx_v.at[j]],
                    rows_v.at[pl.ds(j * CHUNK, CHUNK)],
                    sem,
                )

            @pl.loop(0, n_chunks)
            def _drain(j):
                pltpu.make_async_copy(
                    table_hbm.at[idx_v.at[j]],
                    rows_v.at[pl.ds(j * CHUNK, CHUNK)],
                    sem,
                ).wait()

            pltpu.sync_copy(
                rows_v.at[pl.ds(0, n_chunks * CHUNK)],
                out_hbm.at[pl.ds(wid * ROWS_PER_W, n_chunks * CHUNK)])

        @pl.when(wid < NW - 1)
        def _full():
            run(CHUNKS_PER_W)

        @pl.when(wid == NW - 1)
        def _last():
            run(LAST_CHUNKS)

    return gather_kernel(table, idx2d)


def _featurize_body(t_ref, g_ref, sel_ref, we_ref, s_ref, o_ref):
    t = t_ref[...]
    g2 = g_ref[...]
    g_b = jnp.broadcast_to(g2[:, None, :], (EB // 8, 8, 128)).reshape(EB, 128)
    sel = jnp.broadcast_to(sel_ref[...][None, :, :], (EB // 8, 8, 128)).reshape(
        EB, 128)
    t_b = jnp.broadcast_to(t[:, None, :], (NB, K, 16)).reshape(EB, 16)
    u = jnp.concatenate([g_b * sel, t_b], axis=1)
    te = jax.lax.dot(u, we_ref[...])
    tt = te[:, :192]
    d2w = jax.lax.dot(tt * tt, s_ref[...])
    d = jnp.sqrt(d2w + 1e-6) + DIST_EPS
    m = te[:, 192:193] * te[:, 193:194]
    lanes = jax.lax.broadcasted_iota(jnp.int32, (EB, FDIM), 1)
    feat = jnp.where(lanes < NPAIR, jnp.log(d), 1.0 / d)
    o_ref[...] = feat * m


def _tc_featurize(table: jax.Array, gath: jax.Array) -> jax.Array:
    return pl.pallas_call(
        _featurize_body,
        grid=(GRID,),
        in_specs=[
            pl.BlockSpec((NB, 16), lambda i: (i, 0)),
            pl.BlockSpec((EB // 8, 128), lambda i: (i, 0)),
            pl.BlockSpec((8, 128), lambda i: (0, 0)),
            pl.BlockSpec((144, 256), lambda i: (0, 0)),
            pl.BlockSpec((3 * NPAIR, FDIM), lambda i: (0, 0)),
        ],
        out_specs=pl.BlockSpec((EB, FDIM), lambda i: (i, 0)),
        out_shape=jax.ShapeDtypeStruct((N * K, FDIM), jnp.float32),
        compiler_params=pltpu.CompilerParams(
            dimension_semantics=("arbitrary",),
        ),
    )(table, gath, jnp.asarray(_SEL), jnp.asarray(_WEXT), jnp.asarray(_S))


def kernel(X, edge_idx, C):
    b, n, k = edge_idx.shape
    xf = X.reshape(n, NUM_ATOMS * 3)
    mask = (C.reshape(n) > 0).astype(jnp.float32)
    table = jnp.concatenate(
        [xf, mask[:, None], jnp.zeros((n, 3), jnp.float32)], axis=1)
    table_pad = jnp.pad(table, ((0, 0), (0, 112)))
    idx2d = edge_idx.reshape(n * k // CHUNK, CHUNK).astype(jnp.int32)
    gath = _sc_gather(table_pad.reshape(8 * n, 16), idx2d)
    feat = _tc_featurize(table, gath.reshape(n * k // 8, 128))
    return feat.reshape(b, n, k, FDIM)


# speedup vs baseline: 1.0226x; 1.0154x over previous
import functools

import jax
import jax.numpy as jnp
import numpy as np
from jax import lax
from jax.experimental import pallas as pl
from jax.experimental.pallas import tpu as pltpu
from jax.experimental.pallas import tpu_sc as plsc

N = 10000
K = 16
NUM_ATOMS = 4
NPTS = 2 * NUM_ATOMS
NPAIR = NPTS * NPTS
FDIM = 2 * NPAIR
DIST_EPS = 0.01

NUM_CORES = 2
NUM_SUBCORES = 16
NW = NUM_CORES * NUM_SUBCORES
CHUNK = 128
CHUNKS_PER_W = 40
LAST_CHUNKS = N * K // CHUNK - (NW - 1) * CHUNKS_PER_W
ROWS_PER_W = CHUNKS_PER_W * CHUNK

NB = 200
EB = NB * K
GRID = N // NB


def _pair_matrices():
    wt = np.zeros((16, 3 * NPAIR), np.float32)
    wb = np.zeros((16, 3 * NPAIR), np.float32)
    for p in range(NPTS):
        for q in range(NPTS):
            for c in range(3):
                col = c * NPAIR + p * NPTS + q
                if p < 4:
                    wt[p * 3 + c, col] += 1.0
                else:
                    wb[(p - 4) * 3 + c, col] += 1.0
                if q < 4:
                    wt[q * 3 + c, col] -= 1.0
                else:
                    wb[(q - 4) * 3 + c, col] -= 1.0
    s = np.zeros((3 * NPAIR, FDIM), np.float32)
    for c in range(3):
        for l in range(NPAIR):
            s[c * NPAIR + l, l] = 1.0
            s[c * NPAIR + l, NPAIR + l] = 1.0
    wext = np.zeros((144, 256), np.float32)
    for j in range(8):
        wext[16 * j:16 * j + 16, :192] = wb
        wext[16 * j + 12, 192] = 1.0
    wext[128:144, :192] = wt
    wext[128 + 12, 193] = 1.0
    return wext, s


_WEXT, _S = _pair_matrices()
_SEL = np.zeros((8, 128), np.float32)
for _j in range(8):
    _SEL[_j, 16 * _j:16 * _j + 16] = 1.0


def _sc_gather(table: jax.Array, idx2d: jax.Array) -> jax.Array:
    mesh = plsc.VectorSubcoreMesh(core_axis_name="c", subcore_axis_name="s")

    @functools.partial(
        pl.kernel,
        out_type=jax.ShapeDtypeStruct((N * K, 16), jnp.float32),
        mesh=mesh,
        scratch_types=[
            pltpu.VMEM((CHUNKS_PER_W, 8, CHUNK), jnp.int32),
            pltpu.VMEM((CHUNKS_PER_W, CHUNK), jnp.int32),
            pltpu.VMEM((ROWS_PER_W, 16), jnp.float32),
            pltpu.SemaphoreType.DMA,
        ],
        compiler_params=pltpu.CompilerParams(use_tc_tiling_on_sc=False),
    )
    def gather_kernel(table_hbm, idx_hbm, out_hbm, idx3_v, idx_v, rows_v, sem):
        wid = lax.axis_index("s") * NUM_CORES + lax.axis_index("c")

        def run(n_chunks):
            pltpu.sync_copy(
                idx_hbm.at[pl.ds(wid * CHUNKS_PER_W, n_chunks)],
                idx3_v.at[pl.ds(0, n_chunks)])

            @pl.loop(0, n_chunks)
            def _compact(j):
                for r in range(8):
                    idx_v[j, pl.ds(r * 16, 16)] = idx3_v[j, r, pl.ds(0, 16)] * 8

            @pl.loop(0, n_chunks)
            def _fire(j):
                pltpu.async_copy(
                    table_hbm.at[idx_v.at[j]],
                    rows_v.at[pl.ds(j * CHUNK, CHUNK)],
                    sem,
                )

            @pl.loop(0, n_chunks)
            def _drain(j):
                pltpu.make_async_copy(
                    table_hbm.at[idx_v.at[j]],
                    rows_v.at[pl.ds(j * CHUNK, CHUNK)],
                    sem,
                ).wait()

            pltpu.sync_copy(
                rows_v.at[pl.ds(0, n_chunks * CHUNK)],
                out_hbm.at[pl.ds(wid * ROWS_PER_W, n_chunks * CHUNK)])

        @pl.when(wid < NW - 1)
        def _full():
            run(CHUNKS_PER_W)

        @pl.when(wid == NW - 1)
        def _last():
            run(LAST_CHUNKS)

    return gather_kernel(table, idx2d)


def _featurize_body(t_ref, g_ref, sel_ref, we_ref, s_ref, o_ref):
    t = t_ref[:, :16]
    g2 = g_ref[...]
    g_b = jnp.broadcast_to(g2[:, None, :], (EB // 8, 8, 128)).reshape(EB, 128)
    sel = jnp.broadcast_to(sel_ref[...][None, :, :], (EB // 8, 8, 128)).reshape(
        EB, 128)
    t_b = jnp.broadcast_to(t[:, None, :], (NB, K, 16)).reshape(EB, 16)
    u = jnp.concatenate([g_b * sel, t_b], axis=1)
    te = jax.lax.dot(u, we_ref[...])
    tt = te[:, :192]
    d2w = jax.lax.dot(tt * tt, s_ref[...])
    d = jnp.sqrt(d2w + 1e-6) + DIST_EPS
    m = te[:, 192:193] * te[:, 193:194]
    lanes = jax.lax.broadcasted_iota(jnp.int32, (EB, FDIM), 1)
    feat = jnp.where(lanes < NPAIR, jnp.log(d), 1.0 / d)
    o_ref[...] = feat * m


def _tc_featurize(table: jax.Array, gath: jax.Array) -> jax.Array:
    return pl.pallas_call(
        _featurize_body,
        grid=(GRID,),
        in_specs=[
            pl.BlockSpec((NB, 128), lambda i: (i, 0)),
            pl.BlockSpec((EB // 8, 128), lambda i: (i, 0)),
            pl.BlockSpec((8, 128), lambda i: (0, 0)),
            pl.BlockSpec((144, 256), lambda i: (0, 0)),
            pl.BlockSpec((3 * NPAIR, FDIM), lambda i: (0, 0)),
        ],
        out_specs=pl.BlockSpec((EB, FDIM), lambda i: (i, 0)),
        out_shape=jax.ShapeDtypeStruct((N * K, FDIM), jnp.float32),
        compiler_params=pltpu.CompilerParams(
            dimension_semantics=("arbitrary",),
        ),
    )(table, gath, jnp.asarray(_SEL), jnp.asarray(_WEXT), jnp.asarray(_S))


def kernel(X, edge_idx, C):
    b, n, k = edge_idx.shape
    xf = X.reshape(n, NUM_ATOMS * 3)
    mask = (C.reshape(n) > 0).astype(jnp.float32)
    table = jnp.concatenate(
        [xf, mask[:, None], jnp.zeros((n, 3), jnp.float32)], axis=1)
    table_pad = jnp.pad(table, ((0, 0), (0, 112)))
    idx3 = jnp.pad(edge_idx.astype(jnp.int32).reshape(n, k),
                   ((0, 0), (0, 112))).reshape(n * k // CHUNK, 8, CHUNK)
    gath = _sc_gather(table_pad.reshape(8 * n, 16), idx3)
    feat = _tc_featurize(table_pad, gath.reshape(n * k // 8, 128))
    return feat.reshape(b, n, k, FDIM)


# speedup vs baseline: 1.0411x; 1.0182x over previous
import functools

import jax
import jax.numpy as jnp
import numpy as np
from jax import lax
from jax.experimental import pallas as pl
from jax.experimental.pallas import tpu as pltpu
from jax.experimental.pallas import tpu_sc as plsc

N = 10000
K = 16
NUM_ATOMS = 4
NPTS = 2 * NUM_ATOMS
NPAIR = NPTS * NPTS
FDIM = 2 * NPAIR
DIST_EPS = 0.01

NUM_CORES = 2
NUM_SUBCORES = 16
NW = NUM_CORES * NUM_SUBCORES
CHUNK = 128
HALVES = 2
HCHUNKS = N * K // CHUNK // HALVES
CHUNKS_PER_W = (HCHUNKS + NW - 1) // NW
LAST_CHUNKS = HCHUNKS - (NW - 1) * CHUNKS_PER_W
ROWS_PER_W = CHUNKS_PER_W * CHUNK

NB = 200
EB = NB * K
GRID = N // NB


def _pair_matrices():
    wt = np.zeros((16, 3 * NPAIR), np.float32)
    wb = np.zeros((16, 3 * NPAIR), np.float32)
    for p in range(NPTS):
        for q in range(NPTS):
            for c in range(3):
                col = c * NPAIR + p * NPTS + q
                if p < 4:
                    wt[p * 3 + c, col] += 1.0
                else:
                    wb[(p - 4) * 3 + c, col] += 1.0
                if q < 4:
                    wt[q * 3 + c, col] -= 1.0
                else:
                    wb[(q - 4) * 3 + c, col] -= 1.0
    s = np.zeros((3 * NPAIR, FDIM), np.float32)
    for c in range(3):
        for l in range(NPAIR):
            s[c * NPAIR + l, l] = 1.0
            s[c * NPAIR + l, NPAIR + l] = 1.0
    wext = np.zeros((144, 256), np.float32)
    for j in range(8):
        wext[16 * j:16 * j + 16, :192] = wb
        wext[16 * j + 12, 192] = 1.0
    wext[128:144, :192] = wt
    wext[128 + 12, 193] = 1.0
    return wext, s


_WEXT, _S = _pair_matrices()
_SEL = np.zeros((8, 128), np.float32)
for _j in range(8):
    _SEL[_j, 16 * _j:16 * _j + 16] = 1.0


def _sc_gather(table: jax.Array, idx3: jax.Array, half: int) -> jax.Array:
    mesh = plsc.VectorSubcoreMesh(core_axis_name="c", subcore_axis_name="s")
    base = half * HCHUNKS

    @functools.partial(
        pl.kernel,
        out_type=jax.ShapeDtypeStruct((N * K // HALVES, 16), jnp.float32),
        mesh=mesh,
        scratch_types=[
            pltpu.VMEM((CHUNKS_PER_W, 8, CHUNK), jnp.int32),
            pltpu.VMEM((CHUNKS_PER_W, CHUNK), jnp.int32),
            pltpu.VMEM((ROWS_PER_W, 16), jnp.float32),
            pltpu.SemaphoreType.DMA,
        ],
        compiler_params=pltpu.CompilerParams(use_tc_tiling_on_sc=False),
    )
    def gather_kernel(table_hbm, idx_hbm, out_hbm, idx3_v, idx_v, rows_v, sem):
        wid = lax.axis_index("s") * NUM_CORES + lax.axis_index("c")

        def run(n_chunks):
            pltpu.sync_copy(
                idx_hbm.at[pl.ds(base + wid * CHUNKS_PER_W, n_chunks)],
                idx3_v.at[pl.ds(0, n_chunks)])

            @pl.loop(0, n_chunks)
            def _compact(j):
                for r in range(8):
                    idx_v[j, pl.ds(r * 16, 16)] = idx3_v[j, r, pl.ds(0, 16)] * 8

            @pl.loop(0, n_chunks)
            def _fire(j):
                pltpu.async_copy(
                    table_hbm.at[idx_v.at[j]],
                    rows_v.at[pl.ds(j * CHUNK, CHUNK)],
                    sem,
                )

            @pl.loop(0, n_chunks)
            def _drain(j):
                pltpu.make_async_copy(
                    table_hbm.at[idx_v.at[j]],
                    rows_v.at[pl.ds(j * CHUNK, CHUNK)],
                    sem,
                ).wait()

            pltpu.sync_copy(
                rows_v.at[pl.ds(0, n_chunks * CHUNK)],
                out_hbm.at[pl.ds(wid * ROWS_PER_W, n_chunks * CHUNK)])

        @pl.when(wid < NW - 1)
        def _full():
            run(CHUNKS_PER_W)

        @pl.when(wid == NW - 1)
        def _last():
            run(LAST_CHUNKS)

    return gather_kernel(table, idx3)


def _featurize_body(t_ref, g_ref, sel_ref, we_ref, s_ref, o_ref):
    t = t_ref[:, :16]
    g2 = g_ref[...]
    g_b = jnp.broadcast_to(g2[:, None, :], (EB // 8, 8, 128)).reshape(EB, 128)
    sel = jnp.broadcast_to(sel_ref[...][None, :, :], (EB // 8, 8, 128)).reshape(
        EB, 128)
    t_b = jnp.broadcast_to(t[:, None, :], (NB, K, 16)).reshape(EB, 16)
    u = jnp.concatenate([g_b * sel, t_b], axis=1)
    te = jax.lax.dot(u, we_ref[...])
    tt = te[:, :192]
    d2w = jax.lax.dot(tt * tt, s_ref[...])
    d = jnp.sqrt(d2w + 1e-6) + DIST_EPS
    m = te[:, 192:193] * te[:, 193:194]
    lanes = jax.lax.broadcasted_iota(jnp.int32, (EB, FDIM), 1)
    feat = jnp.where(lanes < NPAIR, jnp.log(d), 1.0 / d)
    o_ref[...] = feat * m


def _featurize_body_alias(t_ref, g_ref, sel_ref, we_ref, s_ref, prev_ref,
                          o_ref):
    del prev_ref
    _featurize_body(t_ref, g_ref, sel_ref, we_ref, s_ref, o_ref)


def _tc_featurize(table: jax.Array, gath: jax.Array, half: int,
                  prev) -> jax.Array:
    hgrid = GRID // HALVES
    off = half * hgrid
    in_specs = [
        pl.BlockSpec((NB, 128), lambda i: (i + off, 0)),
        pl.BlockSpec((EB // 8, 128), lambda i: (i, 0)),
        pl.BlockSpec((8, 128), lambda i: (0, 0)),
        pl.BlockSpec((144, 256), lambda i: (0, 0)),
        pl.BlockSpec((3 * NPAIR, FDIM), lambda i: (0, 0)),
    ]
    args = [table, gath, jnp.asarray(_SEL), jnp.asarray(_WEXT), jnp.asarray(_S)]
    body = _featurize_body
    aliases = {}
    if prev is not None:
        in_specs.append(pl.BlockSpec(memory_space=pl.ANY))
        args.append(prev)
        body = _featurize_body_alias
        aliases = {5: 0}
    return pl.pallas_call(
        body,
        grid=(hgrid,),
        in_specs=in_specs,
        out_specs=pl.BlockSpec((EB, FDIM), lambda i: (i + off, 0)),
        out_shape=jax.ShapeDtypeStruct((N * K, FDIM), jnp.float32),
        input_output_aliases=aliases,
        compiler_params=pltpu.CompilerParams(
            dimension_semantics=("arbitrary",),
        ),
    )(*args)


def kernel(X, edge_idx, C):
    b, n, k = edge_idx.shape
    xf = X.reshape(n, NUM_ATOMS * 3)
    mask = (C.reshape(n) > 0).astype(jnp.float32)
    table = jnp.concatenate(
        [xf, mask[:, None], jnp.zeros((n, 3), jnp.float32)], axis=1)
    table_pad = jnp.pad(table, ((0, 0), (0, 112)))
    idx3 = jnp.pad(edge_idx.astype(jnp.int32),
                   ((0, 0), (0, 0), (0, 112))).reshape(n * k // CHUNK, 8, CHUNK)
    tablev = table_pad.reshape(8 * n, 16)
    feat = None
    for h in range(HALVES):
        gath = _sc_gather(tablev, idx3, h)
        feat = _tc_featurize(table_pad, gath.reshape(n * k // HALVES // 8, 128),
                             h, feat)
    return feat.reshape(b, n, k, FDIM)
